# Initial kernel scaffold; baseline (speedup 1.0000x reference)
import jax
import jax.numpy as jnp
from jax.experimental import pallas as pl


def kernel(x_nchw, skip_nchw, up_w, up_b, c1_w, bn1_g, bn1_b, bn1_m, bn1_v, c2_w, bn2_g, bn2_b, bn2_m, bn2_v):
    raise NotImplementedError("write your pallas kernel here")


# speedup vs baseline: 1.1627x; 1.1627x over previous
import jax
import jax.numpy as jnp
from jax.experimental import pallas as pl
from jax.experimental.pallas import tpu as pltpu

_VMEM_LIMIT = 96 * 1024 * 1024


def _fused_decoder_kernel(x_ref, skip_ref, wup_ref, bup_ref, w1_ref, s1_ref,
                          w2_ref, s2_ref, o_ref):
    H, W, Cin = 32, 32, 256
    C = 128

    x2d = x_ref[...].reshape(H * W, Cin)
    b = bup_ref[...]
    taps = []
    for k in range(4):
        r = jnp.dot(x2d, wup_ref[k], preferred_element_type=jnp.float32) + b
        taps.append(r.reshape(H, W, C).astype(jnp.bfloat16))
    row_even = jnp.stack([taps[0], taps[1]], axis=2).reshape(H, 2 * W, C)
    row_odd = jnp.stack([taps[2], taps[3]], axis=2).reshape(H, 2 * W, C)
    up = jnp.stack([row_even, row_odd], axis=1).reshape(2 * H, 2 * W, C)

    cat = jnp.concatenate([up, skip_ref[0]], axis=-1)
    zrow = jnp.zeros((1, 2 * W, 2 * C), jnp.bfloat16)
    zcol = jnp.zeros((2 * H + 2, 1, 2 * C), jnp.bfloat16)
    xs = jnp.concatenate([zrow, cat, zrow], axis=0)
    xs = jnp.concatenate([zcol, xs, zcol], axis=1)

    M = 2 * H * 2 * W
    acc = jnp.zeros((M, C), jnp.float32)
    for kh in range(3):
        rows = xs[kh:kh + 2 * H]
        a = jnp.concatenate(
            [rows[:, 0:2 * W], rows[:, 1:2 * W + 1], rows[:, 2:2 * W + 2]],
            axis=-1).reshape(M, 3 * 2 * C)
        acc = acc + jnp.dot(a, w1_ref[kh], preferred_element_type=jnp.float32)
    h = jnp.maximum(acc + s1_ref[...], 0.0).astype(jnp.bfloat16)
    h = h.reshape(2 * H, 2 * W, C)

    zrow = jnp.zeros((1, 2 * W, C), jnp.bfloat16)
    zcol = jnp.zeros((2 * H + 2, 1, C), jnp.bfloat16)
    hs = jnp.concatenate([zrow, h, zrow], axis=0)
    hs = jnp.concatenate([zcol, hs, zcol], axis=1)
    acc2 = jnp.zeros((M, C), jnp.float32)
    for kh in range(3):
        rows = hs[kh:kh + 2 * H]
        a = jnp.concatenate(
            [rows[:, 0:2 * W], rows[:, 1:2 * W + 1], rows[:, 2:2 * W + 2]],
            axis=-1).reshape(M, 3 * C)
        acc2 = acc2 + jnp.dot(a, w2_ref[kh], preferred_element_type=jnp.float32)
    y = jnp.maximum(acc2 + s2_ref[...], 0.0)
    o_ref[...] = y.reshape(1, 2 * H, 2 * W, C)


def kernel(x_nchw, skip_nchw, up_w, up_b, c1_w, bn1_g, bn1_b, bn1_m, bn1_v,
           c2_w, bn2_g, bn2_b, bn2_m, bn2_v, *, eps=1e-5):
    N, Cin, H, W = x_nchw.shape
    C = up_w.shape[1]

    x = jnp.transpose(x_nchw, (0, 2, 3, 1)).astype(jnp.bfloat16)
    skip = jnp.transpose(skip_nchw, (0, 2, 3, 1)).astype(jnp.bfloat16)

    wup = jnp.transpose(up_w, (2, 3, 0, 1)).reshape(4, Cin, C)
    wup = wup.astype(jnp.bfloat16)
    bup = up_b.reshape(1, C)

    inv1 = bn1_g / jnp.sqrt(bn1_v + eps)
    inv2 = bn2_g / jnp.sqrt(bn2_v + eps)
    w1 = jnp.transpose(c1_w * inv1[:, None, None, None], (2, 3, 1, 0))
    w1 = w1.reshape(3, 3 * c1_w.shape[1], C).astype(jnp.bfloat16)
    s1 = (bn1_b - bn1_m * inv1).reshape(1, C)
    w2 = jnp.transpose(c2_w * inv2[:, None, None, None], (2, 3, 1, 0))
    w2 = w2.reshape(3, 3 * c2_w.shape[1], C).astype(jnp.bfloat16)
    s2 = (bn2_b - bn2_m * inv2).reshape(1, C)

    out = pl.pallas_call(
        _fused_decoder_kernel,
        out_shape=jax.ShapeDtypeStruct((N, 2 * H, 2 * W, C), jnp.float32),
        grid=(N,),
        in_specs=[
            pl.BlockSpec((1, H, W, Cin), lambda n: (n, 0, 0, 0)),
            pl.BlockSpec((1, 2 * H, 2 * W, C), lambda n: (n, 0, 0, 0)),
            pl.BlockSpec(wup.shape, lambda n: (0, 0, 0)),
            pl.BlockSpec(bup.shape, lambda n: (0, 0)),
            pl.BlockSpec(w1.shape, lambda n: (0, 0, 0)),
            pl.BlockSpec(s1.shape, lambda n: (0, 0)),
            pl.BlockSpec(w2.shape, lambda n: (0, 0, 0)),
            pl.BlockSpec(s2.shape, lambda n: (0, 0)),
        ],
        out_specs=pl.BlockSpec((1, 2 * H, 2 * W, C), lambda n: (n, 0, 0, 0)),
        compiler_params=pltpu.CompilerParams(
            dimension_semantics=("parallel",),
            vmem_limit_bytes=_VMEM_LIMIT,
        ),
    )(x, skip, wup, bup, w1, s1, w2, s2)

    return jnp.transpose(out, (0, 3, 1, 2))
